# Initial kernel scaffold; baseline (speedup 1.0000x reference)
import jax
import jax.numpy as jnp
from jax.experimental import pallas as pl


def kernel(x, edge_index, W_conv, b_conv, W_lin, b_lin):
    raise NotImplementedError("write your pallas kernel here")


# speedup vs baseline: 21.0199x; 21.0199x over previous
import functools

import jax
import jax.numpy as jnp
from jax import lax
from jax.experimental import pallas as pl
from jax.experimental.pallas import tpu as pltpu
from jax.experimental.pallas import tpu_sc as plsc

N = 10000
E = 320000
C = 128

try:
    _info = plsc.get_sparse_core_info()
    NC, NS = _info.num_cores, _info.num_subcores
except Exception:
    NC, NS = 2, 16
NW = NC * NS
EPW = E // NW
NPS = (N // NS) // 8 * 8
NTAIL = N - NPS * NS
K = 80
NCHUNK = EPW // K

_mesh = plsc.VectorSubcoreMesh(core_axis_name="c", subcore_axis_name="s")


@functools.partial(
    pl.kernel,
    out_type=jax.ShapeDtypeStruct((NW, N), jnp.float32),
    mesh=_mesh,
    scratch_types=[
        pltpu.VMEM((EPW,), jnp.int32),
        pltpu.VMEM((N,), jnp.float32),
    ],
    compiler_params=pltpu.CompilerParams(needs_layout_passes=False),
)
def _sc_hist(dst_hbm, zeros_hbm, out_hbm, idx_v, hist_v):
    cid = lax.axis_index("c")
    sid = lax.axis_index("s")
    wid = sid * NC + cid
    pltpu.sync_copy(zeros_hbm, hist_v)
    pltpu.sync_copy(dst_hbm.at[pl.ds(wid * EPW, EPW)], idx_v)
    ones = jnp.ones((16,), jnp.float32)

    def body(i, carry):
        idx = idx_v[pl.ds(i * 16, 16)]
        plsc.addupdate_scatter(hist_v, [idx], ones)
        return carry

    lax.fori_loop(0, EPW // 16, body, 0)
    pltpu.sync_copy(hist_v, out_hbm.at[wid])


@functools.partial(
    pl.kernel,
    out_type=jax.ShapeDtypeStruct((NC, N, C), jnp.float32),
    mesh=_mesh,
    scratch_types=[
        pltpu.VMEM((K,), jnp.int32),
        pltpu.VMEM((K,), jnp.int32),
        pltpu.VMEM((K, C), jnp.float32),
        pltpu.VMEM_SHARED((N, C), jnp.float32),
        pltpu.SemaphoreType.DMA,
    ],
)
def _sc_scatter(src_hbm, dst_hbm, g_hbm, zrows_hbm, out_hbm,
                idx_s, idx_d, rows, acc, sem):
    cid = lax.axis_index("c")
    sid = lax.axis_index("s")
    wid = sid * NC + cid
    pltpu.sync_copy(zrows_hbm, acc.at[pl.ds(sid * NPS, NPS)])

    @pl.when(sid == 0)
    def _():
        pltpu.sync_copy(zrows_hbm.at[pl.ds(0, NTAIL)],
                        acc.at[pl.ds(NPS * NS, NTAIL)])

    plsc.subcore_barrier()

    def body(j, carry):
        base = wid * EPW + j * K
        pltpu.sync_copy(src_hbm.at[pl.ds(base, K)], idx_s)
        pltpu.sync_copy(dst_hbm.at[pl.ds(base, K)], idx_d)
        pltpu.async_copy(g_hbm.at[idx_s], rows, sem).wait()
        pltpu.sync_copy(rows, acc.at[idx_d], add=True)
        return carry

    lax.fori_loop(0, NCHUNK, body, 0)
    plsc.subcore_barrier()
    pltpu.sync_copy(acc.at[pl.ds(sid * NPS, NPS)],
                    out_hbm.at[cid, pl.ds(sid * NPS, NPS)])

    @pl.when(sid == 0)
    def _():
        pltpu.sync_copy(acc.at[pl.ds(NPS * NS, NTAIL)],
                        out_hbm.at[cid, pl.ds(NPS * NS, NTAIL)])


def _tc1_body(parts_ref, x_ref, wc_ref, g_ref, h_ref, dinv_ref):
    deg = jnp.sum(parts_ref[...], axis=0) + 1.0
    dinv = lax.rsqrt(deg)
    h = jnp.dot(x_ref[...], wc_ref[...], preferred_element_type=jnp.float32)
    h_ref[...] = h
    g_ref[...] = h * dinv[:, None]
    dinv_ref[...] = dinv[:, None]


def _tc1(parts, x, wc):
    return pl.pallas_call(
        _tc1_body,
        out_shape=(
            jax.ShapeDtypeStruct((N, C), jnp.float32),
            jax.ShapeDtypeStruct((N, C), jnp.float32),
            jax.ShapeDtypeStruct((N, 1), jnp.float32),
        ),
    )(parts, x, wc)


def _tc2_body(s_ref, h_ref, dinv_ref, bc_ref, wl_ref, bl_ref, out_ref):
    dinv = dinv_ref[...]
    s = s_ref[0] + s_ref[1]
    agg = s * dinv + h_ref[...] * (dinv * dinv) + bc_ref[...]
    out = lax.dot_general(agg, wl_ref[...], (((1,), (1,)), ((), ())),
                          preferred_element_type=jnp.float32)
    out_ref[...] = out + bl_ref[...]


def _tc2(s, h, dinv, bc, wl, bl):
    return pl.pallas_call(
        _tc2_body,
        out_shape=jax.ShapeDtypeStruct((N, C), jnp.float32),
    )(s, h, dinv, bc, wl, bl)


def kernel(x, edge_index, W_conv, b_conv, W_lin, b_lin):
    src = edge_index[0].astype(jnp.int32)
    dst = edge_index[1].astype(jnp.int32)
    zeros_n = jnp.zeros((N,), jnp.float32)
    zrows = jnp.zeros((NPS, C), jnp.float32)
    parts = _sc_hist(dst, zeros_n)
    g, h, dinv = _tc1(parts, x, W_conv)
    s = _sc_scatter(src, dst, g, zrows)
    return _tc2(s, h, dinv, b_conv.reshape(1, C), W_lin, b_lin.reshape(1, C))


# speedup vs baseline: 44.9039x; 2.1363x over previous
import functools

import jax
import jax.numpy as jnp
from jax import lax
from jax.experimental import pallas as pl
from jax.experimental.pallas import tpu as pltpu
from jax.experimental.pallas import tpu_sc as plsc

N = 10000
E = 320000
C = 128

try:
    _info = plsc.get_sparse_core_info()
    NC, NS = _info.num_cores, _info.num_subcores
except Exception:
    NC, NS = 2, 16
NW = NC * NS
EPW = E // NW
NPS = (N // NS) // 8 * 8
NTAIL = N - NPS * NS
K = 100
NCHUNK = EPW // K

_mesh = plsc.VectorSubcoreMesh(core_axis_name="c", subcore_axis_name="s")


@functools.partial(
    pl.kernel,
    out_type=jax.ShapeDtypeStruct((NW, N), jnp.float32),
    mesh=_mesh,
    scratch_types=[
        pltpu.VMEM((EPW,), jnp.int32),
        pltpu.VMEM((N,), jnp.float32),
    ],
    compiler_params=pltpu.CompilerParams(needs_layout_passes=False),
)
def _sc_hist(dst_hbm, zeros_hbm, out_hbm, idx_v, hist_v):
    cid = lax.axis_index("c")
    sid = lax.axis_index("s")
    wid = sid * NC + cid
    pltpu.sync_copy(zeros_hbm, hist_v)
    pltpu.sync_copy(dst_hbm.at[pl.ds(wid * EPW, EPW)], idx_v)
    ones = jnp.ones((16,), jnp.float32)

    def body(i, carry):
        idx = idx_v[pl.ds(i * 16, 16)]
        plsc.addupdate_scatter(hist_v, [idx], ones)
        return carry

    lax.fori_loop(0, EPW // 16, body, 0)
    pltpu.sync_copy(hist_v, out_hbm.at[wid])


@functools.partial(
    pl.kernel,
    out_type=jax.ShapeDtypeStruct((NC, N, C), jnp.float32),
    mesh=_mesh,
    scratch_types=[
        pltpu.VMEM((NCHUNK, K), jnp.int32),
        pltpu.VMEM((2, K), jnp.int32),
        pltpu.VMEM((K, C), jnp.float32),
        pltpu.VMEM((K, C), jnp.float32),
        pltpu.VMEM_SHARED((N, C), jnp.float32),
        pltpu.SemaphoreType.DMA,
        pltpu.SemaphoreType.DMA,
        pltpu.SemaphoreType.DMA,
        pltpu.SemaphoreType.DMA,
    ],
)
def _sc_scatter(src_hbm, dst_hbm, g_hbm, zrows_hbm, out_hbm,
                idx_s, dbuf, rows0, rows1, acc, sem0, sem1, semd0, semd1):
    cid = lax.axis_index("c")
    sid = lax.axis_index("s")
    wid = sid * NC + cid
    pltpu.sync_copy(src_hbm.at[wid], idx_s)
    pltpu.sync_copy(zrows_hbm, acc.at[pl.ds(sid * NPS, NPS)])

    @pl.when(sid == 0)
    def _():
        pltpu.sync_copy(zrows_hbm.at[pl.ds(0, NTAIL)],
                        acc.at[pl.ds(NPS * NS, NTAIL)])

    plsc.subcore_barrier()

    rows = (rows0, rows1)
    sems = (sem0, sem1)
    semd = (semd0, semd1)
    pltpu.async_copy(g_hbm.at[idx_s.at[0]], rows0, sem0)
    pltpu.async_copy(dst_hbm.at[wid, 0], dbuf.at[0], semd0)

    @pl.loop(0, NCHUNK, step=2)
    def _(j):
        for b in range(2):
            jj = j + b

            @pl.when(jj + 1 < NCHUNK)
            def _():
                pltpu.async_copy(g_hbm.at[idx_s.at[jj + 1]],
                                 rows[1 - b], sems[1 - b])
                pltpu.async_copy(dst_hbm.at[wid, jj + 1],
                                 dbuf.at[1 - b], semd[1 - b])

            pltpu.make_async_copy(g_hbm.at[idx_s.at[jj]],
                                  rows[b], sems[b]).wait()
            pltpu.make_async_copy(dst_hbm.at[wid, jj],
                                  dbuf.at[b], semd[b]).wait()
            pltpu.sync_copy(rows[b], acc.at[dbuf.at[b]], add=True)

    plsc.subcore_barrier()
    pltpu.sync_copy(acc.at[pl.ds(sid * NPS, NPS)],
                    out_hbm.at[cid, pl.ds(sid * NPS, NPS)])

    @pl.when(sid == 0)
    def _():
        pltpu.sync_copy(acc.at[pl.ds(NPS * NS, NTAIL)],
                        out_hbm.at[cid, pl.ds(NPS * NS, NTAIL)])


def _tc1_body(parts_ref, x_ref, wc_ref, g_ref, h_ref, dinv_ref):
    deg = jnp.sum(parts_ref[...], axis=0) + 1.0
    dinv = lax.rsqrt(deg)
    h = jnp.dot(x_ref[...], wc_ref[...], preferred_element_type=jnp.float32)
    h_ref[...] = h
    g_ref[...] = h * dinv[:, None]
    dinv_ref[...] = dinv[:, None]


def _tc1(parts, x, wc):
    return pl.pallas_call(
        _tc1_body,
        out_shape=(
            jax.ShapeDtypeStruct((N, C), jnp.float32),
            jax.ShapeDtypeStruct((N, C), jnp.float32),
            jax.ShapeDtypeStruct((N, 1), jnp.float32),
        ),
    )(parts, x, wc)


def _tc2_body(s_ref, h_ref, dinv_ref, bc_ref, wl_ref, bl_ref, out_ref):
    dinv = dinv_ref[...]
    s = s_ref[0] + s_ref[1]
    agg = s * dinv + h_ref[...] * (dinv * dinv) + bc_ref[...]
    out = lax.dot_general(agg, wl_ref[...], (((1,), (1,)), ((), ())),
                          preferred_element_type=jnp.float32)
    out_ref[...] = out + bl_ref[...]


def _tc2(s, h, dinv, bc, wl, bl):
    return pl.pallas_call(
        _tc2_body,
        out_shape=jax.ShapeDtypeStruct((N, C), jnp.float32),
    )(s, h, dinv, bc, wl, bl)


def kernel(x, edge_index, W_conv, b_conv, W_lin, b_lin):
    src = edge_index[0].astype(jnp.int32)
    dst = edge_index[1].astype(jnp.int32)
    zeros_n = jnp.zeros((N,), jnp.float32)
    zrows = jnp.zeros((NPS, C), jnp.float32)
    parts = _sc_hist(dst, zeros_n)
    g, h, dinv = _tc1(parts, x, W_conv)
    src3 = src.reshape(NW, NCHUNK, K)
    dst3 = dst.reshape(NW, NCHUNK, K)
    s = _sc_scatter(src3, dst3, g, zrows)
    return _tc2(s, h, dinv, b_conv.reshape(1, C), W_lin, b_lin.reshape(1, C))


# speedup vs baseline: 45.8992x; 1.0222x over previous
import functools

import jax
import jax.numpy as jnp
from jax import lax
from jax.experimental import pallas as pl
from jax.experimental.pallas import tpu as pltpu
from jax.experimental.pallas import tpu_sc as plsc

N = 10000
E = 320000
C = 128

try:
    _info = plsc.get_sparse_core_info()
    NC, NS = _info.num_cores, _info.num_subcores
except Exception:
    NC, NS = 2, 16
NW = NC * NS
EPW = E // NW
NPS = (N // NS) // 8 * 8
NTAIL = N - NPS * NS
K = 80
NCHUNK = EPW // K

_mesh = plsc.VectorSubcoreMesh(core_axis_name="c", subcore_axis_name="s")


@functools.partial(
    pl.kernel,
    out_type=jax.ShapeDtypeStruct((NW, N), jnp.float32),
    mesh=_mesh,
    scratch_types=[
        pltpu.VMEM((EPW,), jnp.int32),
        pltpu.VMEM((N,), jnp.float32),
    ],
    compiler_params=pltpu.CompilerParams(needs_layout_passes=False),
)
def _sc_hist(ei_hbm, out_hbm, idx_v, hist_v):
    cid = lax.axis_index("c")
    sid = lax.axis_index("s")
    wid = sid * NC + cid
    pltpu.sync_copy(ei_hbm.at[pl.ds(E + wid * EPW, EPW)], idx_v)

    def zbody(i, carry):
        hist_v[pl.ds(i * 16, 16)] = jnp.zeros((16,), jnp.float32)
        return carry

    lax.fori_loop(0, N // 16, zbody, 0)
    ones = jnp.ones((16,), jnp.float32)

    def body(i, carry):
        idx = idx_v[pl.ds(i * 16, 16)]
        plsc.addupdate_scatter(hist_v, [idx], ones)
        return carry

    lax.fori_loop(0, EPW // 16, body, 0)
    pltpu.sync_copy(hist_v, out_hbm.at[wid])


@functools.partial(
    pl.kernel,
    out_type=jax.ShapeDtypeStruct((NC, N, C), jnp.float32),
    mesh=_mesh,
    scratch_types=[
        pltpu.VMEM((EPW,), jnp.int32),
        pltpu.VMEM((2, K), jnp.int32),
        pltpu.VMEM((K, C), jnp.float32),
        pltpu.VMEM((K, C), jnp.float32),
        pltpu.VMEM_SHARED((N, C), jnp.float32),
        pltpu.SemaphoreType.DMA,
        pltpu.SemaphoreType.DMA,
        pltpu.SemaphoreType.DMA,
        pltpu.SemaphoreType.DMA,
        pltpu.SemaphoreType.DMA,
        pltpu.SemaphoreType.DMA,
    ],
)
def _sc_scatter(ei_hbm, g_hbm, zrows_hbm, out_hbm,
                idx_s, dbuf, rows0, rows1, acc,
                sem0, sem1, semd0, semd1, semw0, semw1):
    cid = lax.axis_index("c")
    sid = lax.axis_index("s")
    wid = sid * NC + cid
    sbase = wid * EPW
    dbase = E + wid * EPW
    pltpu.sync_copy(ei_hbm.at[pl.ds(sbase, EPW)], idx_s)
    pltpu.sync_copy(zrows_hbm, acc.at[pl.ds(sid * NPS, NPS)])

    @pl.when(sid == 0)
    def _():
        pltpu.sync_copy(zrows_hbm.at[pl.ds(0, NTAIL)],
                        acc.at[pl.ds(NPS * NS, NTAIL)])

    plsc.subcore_barrier()

    rows = (rows0, rows1)
    sems = (sem0, sem1)
    semd = (semd0, semd1)
    semw = (semw0, semw1)

    def start_gather(jj, b):
        pltpu.async_copy(g_hbm.at[idx_s.at[pl.ds(jj * K, K)]], rows[b], sems[b])
        pltpu.async_copy(ei_hbm.at[pl.ds(dbase + jj * K, K)], dbuf.at[b], semd[b])

    def wait_gather(jj, b):
        pltpu.make_async_copy(g_hbm.at[idx_s.at[pl.ds(jj * K, K)]],
                              rows[b], sems[b]).wait()
        pltpu.make_async_copy(ei_hbm.at[pl.ds(dbase + jj * K, K)],
                              dbuf.at[b], semd[b]).wait()

    def start_scatter(b):
        pltpu.async_copy(rows[b], acc.at[dbuf.at[b]], semw[b], add=True)

    def wait_scatter(b):
        pltpu.make_async_copy(rows[b], acc.at[dbuf.at[b]], semw[b]).wait()

    start_gather(0, 0)

    @pl.loop(0, NCHUNK - 1, step=2)
    def _(j):
        for b in range(2):
            jj = j + b

            @pl.when(jj >= 1)
            def _():
                wait_scatter(1 - b)
            start_gather(jj + 1, 1 - b)
            wait_gather(jj, b)
            start_scatter(b)

    wait_scatter(1)
    wait_gather(NCHUNK - 1, 0)
    pltpu.sync_copy(rows[0], acc.at[dbuf.at[0]], add=True)
    plsc.subcore_barrier()
    pltpu.sync_copy(acc.at[pl.ds(sid * NPS, NPS)],
                    out_hbm.at[cid, pl.ds(sid * NPS, NPS)])

    @pl.when(sid == 0)
    def _():
        pltpu.sync_copy(acc.at[pl.ds(NPS * NS, NTAIL)],
                        out_hbm.at[cid, pl.ds(NPS * NS, NTAIL)])


def _tc1_body(parts_ref, x_ref, wc_ref, g_ref, h_ref, dinv_ref):
    deg = jnp.sum(parts_ref[...], axis=0) + 1.0
    dinv = lax.rsqrt(deg)
    h = jnp.dot(x_ref[...], wc_ref[...], preferred_element_type=jnp.float32)
    h_ref[...] = h
    g_ref[...] = h * dinv[:, None]
    dinv_ref[...] = dinv[:, None]


def _tc1(parts, x, wc):
    return pl.pallas_call(
        _tc1_body,
        out_shape=(
            jax.ShapeDtypeStruct((N, C), jnp.float32),
            jax.ShapeDtypeStruct((N, C), jnp.float32),
            jax.ShapeDtypeStruct((N, 1), jnp.float32),
        ),
    )(parts, x, wc)


def _tc2_body(s_ref, h_ref, dinv_ref, bc_ref, wl_ref, bl_ref, out_ref):
    dinv = dinv_ref[...]
    s = s_ref[0] + s_ref[1]
    agg = s * dinv + h_ref[...] * (dinv * dinv) + bc_ref[...]
    out = lax.dot_general(agg, wl_ref[...], (((1,), (1,)), ((), ())),
                          preferred_element_type=jnp.float32)
    out_ref[...] = out + bl_ref[...]


def _tc2(s, h, dinv, bc, wl, bl):
    return pl.pallas_call(
        _tc2_body,
        out_shape=jax.ShapeDtypeStruct((N, C), jnp.float32),
    )(s, h, dinv, bc, wl, bl)


def kernel(x, edge_index, W_conv, b_conv, W_lin, b_lin):
    ei = edge_index.reshape(2 * E).astype(jnp.int32)
    zrows = jnp.zeros((NPS, C), jnp.float32)
    parts = _sc_hist(ei)
    g, h, dinv = _tc1(parts, x, W_conv)
    s = _sc_scatter(ei, g, zrows)
    return _tc2(s, h, dinv, b_conv.reshape(1, C), W_lin, b_lin.reshape(1, C))


# speedup vs baseline: 49.9182x; 1.0876x over previous
import functools

import jax
import jax.numpy as jnp
from jax import lax
from jax.experimental import pallas as pl
from jax.experimental.pallas import tpu as pltpu
from jax.experimental.pallas import tpu_sc as plsc

N = 10000
E = 320000
C = 128

try:
    _info = plsc.get_sparse_core_info()
    NC, NS = _info.num_cores, _info.num_subcores
except Exception:
    NC, NS = 2, 16
NW = NC * NS
K = 128
CHUNKS = E // K
CPW = CHUNKS // NW
EXTRA = CHUNKS - CPW * NW
NPS = (N // NS) // 8 * 8
NTAIL = N - NPS * NS

_mesh = plsc.VectorSubcoreMesh(core_axis_name="c", subcore_axis_name="s")


@functools.partial(
    pl.kernel,
    out_type=jax.ShapeDtypeStruct((NW, N), jnp.float32),
    mesh=_mesh,
    scratch_types=[
        pltpu.VMEM((2, CPW * K), jnp.int32),
        pltpu.VMEM((2, K), jnp.int32),
        pltpu.VMEM((N,), jnp.float32),
    ],
    compiler_params=pltpu.CompilerParams(needs_layout_passes=False),
)
def _sc_hist(ei_hbm, out_hbm, ebuf, ebuf_x, hist_v):
    cid = lax.axis_index("c")
    sid = lax.axis_index("s")
    wid = sid * NC + cid
    pltpu.sync_copy(ei_hbm.at[:, pl.ds(wid * CPW * K, CPW * K)], ebuf)

    def zbody(i, carry):
        hist_v[pl.ds(i * 16, 16)] = jnp.zeros((16,), jnp.float32)
        return carry

    lax.fori_loop(0, N // 16, zbody, 0)
    ones = jnp.ones((16,), jnp.float32)

    def body(i, carry):
        idx = ebuf[1, pl.ds(i * 16, 16)]
        plsc.addupdate_scatter(hist_v, [idx], ones)
        return carry

    lax.fori_loop(0, CPW * K // 16, body, 0)

    @pl.when(wid < EXTRA)
    def _():
        pltpu.sync_copy(ei_hbm.at[:, pl.ds((CPW * NW + wid) * K, K)], ebuf_x)

        def xbody(i, carry):
            idx = ebuf_x[1, pl.ds(i * 16, 16)]
            plsc.addupdate_scatter(hist_v, [idx], ones)
            return carry

        lax.fori_loop(0, K // 16, xbody, 0)

    pltpu.sync_copy(hist_v, out_hbm.at[wid])


@functools.partial(
    pl.kernel,
    out_type=jax.ShapeDtypeStruct((NC, N, C), jnp.float32),
    mesh=_mesh,
    scratch_types=[
        pltpu.VMEM((2, K), jnp.int32),
        pltpu.VMEM((2, K), jnp.int32),
        pltpu.VMEM((2, K), jnp.int32),
        pltpu.VMEM((K, C), jnp.float32),
        pltpu.VMEM((K, C), jnp.float32),
        pltpu.VMEM_SHARED((N, C), jnp.float32),
        pltpu.SemaphoreType.DMA,
        pltpu.SemaphoreType.DMA,
        pltpu.SemaphoreType.DMA,
        pltpu.SemaphoreType.DMA,
        pltpu.SemaphoreType.DMA,
    ],
)
def _sc_scatter(ei_hbm, g_hbm, zrows_hbm, out_hbm,
                eb0, eb1, eb2, rows0, rows1, acc,
                semE0, semE1, semE2, semG0, semG1):
    cid = lax.axis_index("c")
    sid = lax.axis_index("s")
    wid = sid * NC + cid
    cbase = wid * CPW
    pltpu.sync_copy(zrows_hbm, acc.at[pl.ds(sid * NPS, NPS)])

    @pl.when(sid == 0)
    def _():
        pltpu.sync_copy(zrows_hbm.at[pl.ds(0, NTAIL)],
                        acc.at[pl.ds(NPS * NS, NTAIL)])

    plsc.subcore_barrier()

    ebuf = (eb0, eb1, eb2)
    semE = (semE0, semE1, semE2)
    rows = (rows0, rows1)
    semG = (semG0, semG1)

    def eload_start(r, e):
        pltpu.async_copy(ei_hbm.at[:, pl.ds((cbase + r) * K, K)],
                         ebuf[e], semE[e])

    def eload_wait(r, e):
        pltpu.make_async_copy(ei_hbm.at[:, pl.ds((cbase + r) * K, K)],
                              ebuf[e], semE[e]).wait()

    def gather_start(e, b):
        pltpu.async_copy(g_hbm.at[ebuf[e].at[0]], rows[b], semG[b])

    def gather_wait(e, b):
        pltpu.make_async_copy(g_hbm.at[ebuf[e].at[0]], rows[b], semG[b]).wait()

    def scatter(e, b):
        pltpu.sync_copy(rows[b], acc.at[ebuf[e].at[1]], add=True)

    eload_start(0, 0)
    eload_start(1, 1)
    eload_start(2, 2)
    eload_wait(0, 0)
    gather_start(0, 0)

    @pl.loop(0, CPW, step=6)
    def _(j):
        for u in range(6):
            r = j + u
            e = u % 3
            b = u % 2
            e1 = (u + 1) % 3

            @pl.when(r + 1 < CPW)
            def _():
                eload_wait(r + 1, e1)
                gather_start(e1, 1 - b)

            gather_wait(e, b)
            scatter(e, b)

            @pl.when(r + 3 < CPW)
            def _():
                eload_start(r + 3, e)

    @pl.when(wid < EXTRA)
    def _():
        pltpu.sync_copy(ei_hbm.at[:, pl.ds((CPW * NW + wid) * K, K)], eb0)
        pltpu.async_copy(g_hbm.at[eb0.at[0]], rows0, semG0)
        pltpu.make_async_copy(g_hbm.at[eb0.at[0]], rows0, semG0).wait()
        pltpu.sync_copy(rows0, acc.at[eb0.at[1]], add=True)

    plsc.subcore_barrier()
    pltpu.sync_copy(acc.at[pl.ds(sid * NPS, NPS)],
                    out_hbm.at[cid, pl.ds(sid * NPS, NPS)])

    @pl.when(sid == 0)
    def _():
        pltpu.sync_copy(acc.at[pl.ds(NPS * NS, NTAIL)],
                        out_hbm.at[cid, pl.ds(NPS * NS, NTAIL)])


def _tc1_body(parts_ref, x_ref, wc_ref, g_ref, h_ref, dinv_ref):
    deg = jnp.sum(parts_ref[...], axis=0) + 1.0
    dinv = lax.rsqrt(deg)
    h = jnp.dot(x_ref[...], wc_ref[...], preferred_element_type=jnp.float32)
    h_ref[...] = h
    g_ref[...] = h * dinv[:, None]
    dinv_ref[...] = dinv[:, None]


def _tc1(parts, x, wc):
    return pl.pallas_call(
        _tc1_body,
        out_shape=(
            jax.ShapeDtypeStruct((N, C), jnp.float32),
            jax.ShapeDtypeStruct((N, C), jnp.float32),
            jax.ShapeDtypeStruct((N, 1), jnp.float32),
        ),
    )(parts, x, wc)


def _tc2_body(s_ref, h_ref, dinv_ref, bc_ref, wl_ref, bl_ref, out_ref):
    dinv = dinv_ref[...]
    s = s_ref[0] + s_ref[1]
    agg = s * dinv + h_ref[...] * (dinv * dinv) + bc_ref[...]
    out = lax.dot_general(agg, wl_ref[...], (((1,), (1,)), ((), ())),
                          preferred_element_type=jnp.float32)
    out_ref[...] = out + bl_ref[...]


def _tc2(s, h, dinv, bc, wl, bl):
    return pl.pallas_call(
        _tc2_body,
        out_shape=jax.ShapeDtypeStruct((N, C), jnp.float32),
    )(s, h, dinv, bc, wl, bl)


def kernel(x, edge_index, W_conv, b_conv, W_lin, b_lin):
    ei = edge_index.astype(jnp.int32)
    zrows = jnp.zeros((NPS, C), jnp.float32)
    parts = _sc_hist(ei)
    g, h, dinv = _tc1(parts, x, W_conv)
    s = _sc_scatter(ei, g, zrows)
    return _tc2(s, h, dinv, b_conv.reshape(1, C), W_lin, b_lin.reshape(1, C))


# speedup vs baseline: 51.8385x; 1.0385x over previous
import functools

import jax
import jax.numpy as jnp
from jax import lax
from jax.experimental import pallas as pl
from jax.experimental.pallas import tpu as pltpu
from jax.experimental.pallas import tpu_sc as plsc

N = 10000
E = 320000
C = 128

try:
    _info = plsc.get_sparse_core_info()
    NC, NS = _info.num_cores, _info.num_subcores
except Exception:
    NC, NS = 2, 16
NW = NC * NS
K = 128
CHUNKS = E // K
CPW = CHUNKS // NW
EXTRA = CHUNKS - CPW * NW
NPS = (N // NS) // 8 * 8
NTAIL = N - NPS * NS

_mesh = plsc.VectorSubcoreMesh(core_axis_name="c", subcore_axis_name="s")


@functools.partial(
    pl.kernel,
    out_type=jax.ShapeDtypeStruct((NW, N), jnp.float32),
    mesh=_mesh,
    scratch_types=[
        pltpu.VMEM((2, CPW * K), jnp.int32),
        pltpu.VMEM((2, K), jnp.int32),
        pltpu.VMEM((N,), jnp.float32),
        pltpu.SemaphoreType.DMA,
    ],
    compiler_params=pltpu.CompilerParams(needs_layout_passes=False),
)
def _sc_hist(ei_hbm, out_hbm, ebuf, ebuf_x, hist_v, sem):
    cid = lax.axis_index("c")
    sid = lax.axis_index("s")
    wid = sid * NC + cid
    cp = pltpu.async_copy(ei_hbm.at[:, pl.ds(wid * CPW * K, CPW * K)],
                          ebuf, sem)

    def zbody(i, carry):
        hist_v[pl.ds(i * 16, 16)] = jnp.zeros((16,), jnp.float32)
        return carry

    lax.fori_loop(0, N // 16, zbody, 0)
    cp.wait()
    ones = jnp.ones((16,), jnp.float32)

    def body(i, carry):
        idx = ebuf[1, pl.ds(i * 16, 16)]
        plsc.addupdate_scatter(hist_v, [idx], ones)
        return carry

    lax.fori_loop(0, CPW * K // 16, body, 0)

    @pl.when(wid < EXTRA)
    def _():
        pltpu.sync_copy(ei_hbm.at[:, pl.ds((CPW * NW + wid) * K, K)], ebuf_x)

        def xbody(i, carry):
            idx = ebuf_x[1, pl.ds(i * 16, 16)]
            plsc.addupdate_scatter(hist_v, [idx], ones)
            return carry

        lax.fori_loop(0, K // 16, xbody, 0)

    pltpu.sync_copy(hist_v, out_hbm.at[wid])


@functools.partial(
    pl.kernel,
    out_type=jax.ShapeDtypeStruct((NC, N, C), jnp.float32),
    mesh=_mesh,
    scratch_types=[
        pltpu.VMEM((2, K), jnp.int32),
        pltpu.VMEM((2, K), jnp.int32),
        pltpu.VMEM((2, K), jnp.int32),
        pltpu.VMEM((K, C), jnp.float32),
        pltpu.VMEM((K, C), jnp.float32),
        pltpu.VMEM_SHARED((N, C), jnp.float32),
        pltpu.SemaphoreType.DMA,
        pltpu.SemaphoreType.DMA,
        pltpu.SemaphoreType.DMA,
        pltpu.SemaphoreType.DMA,
        pltpu.SemaphoreType.DMA,
    ],
)
def _sc_scatter(ei_hbm, g_hbm, out_hbm,
                eb0, eb1, eb2, rows0, rows1, acc,
                semE0, semE1, semE2, semG0, semG1):
    cid = lax.axis_index("c")
    sid = lax.axis_index("s")
    wid = sid * NC + cid
    cbase = wid * CPW

    def zr(r, carry):
        def zc(c, carry2):
            rows0[r, pl.ds(c * 16, 16)] = jnp.zeros((16,), jnp.float32)
            return carry2
        return lax.fori_loop(0, C // 16, zc, carry)

    lax.fori_loop(0, K, zr, 0)
    for q in range(NPS // K):
        pltpu.sync_copy(rows0, acc.at[pl.ds(sid * NPS + q * K, K)])
    pltpu.sync_copy(rows0.at[pl.ds(0, NPS - (NPS // K) * K)],
                    acc.at[pl.ds(sid * NPS + (NPS // K) * K,
                                 NPS - (NPS // K) * K)])

    @pl.when(sid == 0)
    def _():
        pltpu.sync_copy(rows0.at[pl.ds(0, NTAIL)],
                        acc.at[pl.ds(NPS * NS, NTAIL)])

    plsc.subcore_barrier()

    ebuf = (eb0, eb1, eb2)
    semE = (semE0, semE1, semE2)
    rows = (rows0, rows1)
    semG = (semG0, semG1)

    def eload_start(r, e):
        pltpu.async_copy(ei_hbm.at[:, pl.ds((cbase + r) * K, K)],
                         ebuf[e], semE[e])

    def eload_wait(r, e):
        pltpu.make_async_copy(ei_hbm.at[:, pl.ds((cbase + r) * K, K)],
                              ebuf[e], semE[e]).wait()

    def gather_start(e, b):
        pltpu.async_copy(g_hbm.at[ebuf[e].at[0]], rows[b], semG[b])

    def gather_wait(e, b):
        pltpu.make_async_copy(g_hbm.at[ebuf[e].at[0]], rows[b], semG[b]).wait()

    def scatter(e, b):
        pltpu.sync_copy(rows[b], acc.at[ebuf[e].at[1]], add=True)

    eload_start(0, 0)
    eload_start(1, 1)
    eload_start(2, 2)
    eload_wait(0, 0)
    gather_start(0, 0)

    @pl.loop(0, CPW, step=6)
    def _(j):
        for u in range(6):
            r = j + u
            e = u % 3
            b = u % 2
            e1 = (u + 1) % 3

            @pl.when(r + 1 < CPW)
            def _():
                eload_wait(r + 1, e1)
                gather_start(e1, 1 - b)

            gather_wait(e, b)
            scatter(e, b)

            @pl.when(r + 3 < CPW)
            def _():
                eload_start(r + 3, e)

    @pl.when(wid < EXTRA)
    def _():
        pltpu.sync_copy(ei_hbm.at[:, pl.ds((CPW * NW + wid) * K, K)], eb0)
        pltpu.async_copy(g_hbm.at[eb0.at[0]], rows0, semG0)
        pltpu.make_async_copy(g_hbm.at[eb0.at[0]], rows0, semG0).wait()
        pltpu.sync_copy(rows0, acc.at[eb0.at[1]], add=True)

    plsc.subcore_barrier()
    pltpu.sync_copy(acc.at[pl.ds(sid * NPS, NPS)],
                    out_hbm.at[cid, pl.ds(sid * NPS, NPS)])

    @pl.when(sid == 0)
    def _():
        pltpu.sync_copy(acc.at[pl.ds(NPS * NS, NTAIL)],
                        out_hbm.at[cid, pl.ds(NPS * NS, NTAIL)])


_RB = 1000
_GRID = N // _RB


def _tc1a_body(x_ref, wc_ref, h_ref):
    h_ref[...] = jnp.dot(x_ref[...], wc_ref[...],
                         preferred_element_type=jnp.float32)


def _tc1a(x, wc):
    return pl.pallas_call(
        _tc1a_body,
        grid=(_GRID,),
        in_specs=[
            pl.BlockSpec((_RB, C), lambda i: (i, 0)),
            pl.BlockSpec((C, C), lambda i: (0, 0)),
        ],
        out_specs=pl.BlockSpec((_RB, C), lambda i: (i, 0)),
        out_shape=jax.ShapeDtypeStruct((N, C), jnp.float32),
    )(x, wc)


def _tc1b_body(parts_ref, h_ref, g_ref, dinv_ref):
    deg = jnp.sum(parts_ref[...], axis=0) + 1.0
    dinv = lax.rsqrt(deg)
    g_ref[...] = h_ref[...] * dinv[:, None]
    dinv_ref[...] = dinv[:, None]


def _tc1b(parts, h):
    return pl.pallas_call(
        _tc1b_body,
        out_shape=(
            jax.ShapeDtypeStruct((N, C), jnp.float32),
            jax.ShapeDtypeStruct((N, 1), jnp.float32),
        ),
    )(parts, h)


def _tc2_body(s_ref, g_ref, dinv_ref, bc_ref, wl_ref, bl_ref, out_ref):
    dinv = dinv_ref[...]
    agg = (s_ref[0] + s_ref[1] + g_ref[...]) * dinv + bc_ref[...]
    out = lax.dot_general(agg, wl_ref[...], (((1,), (1,)), ((), ())),
                          preferred_element_type=jnp.float32)
    out_ref[...] = out + bl_ref[...]


def _tc2(s, g, dinv, bc, wl, bl):
    return pl.pallas_call(
        _tc2_body,
        grid=(_GRID,),
        in_specs=[
            pl.BlockSpec((2, _RB, C), lambda i: (0, i, 0)),
            pl.BlockSpec((_RB, C), lambda i: (i, 0)),
            pl.BlockSpec((_RB, 1), lambda i: (i, 0)),
            pl.BlockSpec((1, C), lambda i: (0, 0)),
            pl.BlockSpec((C, C), lambda i: (0, 0)),
            pl.BlockSpec((1, C), lambda i: (0, 0)),
        ],
        out_specs=pl.BlockSpec((_RB, C), lambda i: (i, 0)),
        out_shape=jax.ShapeDtypeStruct((N, C), jnp.float32),
    )(s, g, dinv, bc, wl, bl)


def kernel(x, edge_index, W_conv, b_conv, W_lin, b_lin):
    ei = edge_index.astype(jnp.int32)
    parts = _sc_hist(ei)
    h = _tc1a(x, W_conv)
    g, dinv = _tc1b(parts, h)
    s = _sc_scatter(ei, g)
    return _tc2(s, g, dinv, b_conv.reshape(1, C), W_lin, b_lin.reshape(1, C))


# speedup vs baseline: 52.5929x; 1.0146x over previous
import functools

import jax
import jax.numpy as jnp
from jax import lax
from jax.experimental import pallas as pl
from jax.experimental.pallas import tpu as pltpu
from jax.experimental.pallas import tpu_sc as plsc

N = 10000
E = 320000
C = 128

try:
    _info = plsc.get_sparse_core_info()
    NC, NS = _info.num_cores, _info.num_subcores
except Exception:
    NC, NS = 2, 16
NW = NC * NS
K = 128
CHUNKS = E // K
CPW = CHUNKS // NW
EXTRA = CHUNKS - CPW * NW
NPS = (N // NS) // 8 * 8
NTAIL = N - NPS * NS

_mesh = plsc.VectorSubcoreMesh(core_axis_name="c", subcore_axis_name="s")


@functools.partial(
    pl.kernel,
    out_type=jax.ShapeDtypeStruct((NW, N), jnp.float32),
    mesh=_mesh,
    scratch_types=[
        pltpu.VMEM((2, CPW * K), jnp.int32),
        pltpu.VMEM((2, K), jnp.int32),
        pltpu.VMEM((N,), jnp.float32),
        pltpu.SemaphoreType.DMA,
    ],
    compiler_params=pltpu.CompilerParams(needs_layout_passes=False),
)
def _sc_hist(ei_hbm, out_hbm, ebuf, ebuf_x, hist_v, sem):
    cid = lax.axis_index("c")
    sid = lax.axis_index("s")
    wid = sid * NC + cid
    cp = pltpu.async_copy(ei_hbm.at[:, pl.ds(wid * CPW * K, CPW * K)],
                          ebuf, sem)

    def zbody(i, carry):
        hist_v[pl.ds(i * 16, 16)] = jnp.zeros((16,), jnp.float32)
        return carry

    lax.fori_loop(0, N // 16, zbody, 0)
    cp.wait()
    ones = jnp.ones((16,), jnp.float32)

    def body(i, carry):
        idx = ebuf[1, pl.ds(i * 16, 16)]
        plsc.addupdate_scatter(hist_v, [idx], ones)
        return carry

    lax.fori_loop(0, CPW * K // 16, body, 0)

    @pl.when(wid < EXTRA)
    def _():
        pltpu.sync_copy(ei_hbm.at[:, pl.ds((CPW * NW + wid) * K, K)], ebuf_x)

        def xbody(i, carry):
            idx = ebuf_x[1, pl.ds(i * 16, 16)]
            plsc.addupdate_scatter(hist_v, [idx], ones)
            return carry

        lax.fori_loop(0, K // 16, xbody, 0)

    pltpu.sync_copy(hist_v, out_hbm.at[wid])


@functools.partial(
    pl.kernel,
    out_type=jax.ShapeDtypeStruct((NC, N, C), jnp.bfloat16),
    mesh=_mesh,
    scratch_types=[
        pltpu.VMEM((2, K), jnp.int32),
        pltpu.VMEM((2, K), jnp.int32),
        pltpu.VMEM((2, K), jnp.int32),
        pltpu.VMEM((K, C), jnp.bfloat16),
        pltpu.VMEM((K, C), jnp.bfloat16),
        pltpu.VMEM_SHARED((N, C), jnp.bfloat16),
        pltpu.SemaphoreType.DMA,
        pltpu.SemaphoreType.DMA,
        pltpu.SemaphoreType.DMA,
        pltpu.SemaphoreType.DMA,
        pltpu.SemaphoreType.DMA,
    ],
    compiler_params=pltpu.CompilerParams(use_tc_tiling_on_sc=False),
)
def _sc_scatter(ei_hbm, g_hbm, out_hbm,
                eb0, eb1, eb2, rows0, rows1, acc,
                semE0, semE1, semE2, semG0, semG1):
    cid = lax.axis_index("c")
    sid = lax.axis_index("s")
    wid = sid * NC + cid
    cbase = wid * CPW

    def zr(r, carry):
        def zc(c, carry2):
            rows0[r, pl.ds(c * 32, 32)] = jnp.zeros((32,), jnp.bfloat16)
            return carry2
        return lax.fori_loop(0, C // 32, zc, carry)

    lax.fori_loop(0, K, zr, 0)
    for q in range(NPS // K):
        pltpu.sync_copy(rows0, acc.at[pl.ds(sid * NPS + q * K, K)])
    pltpu.sync_copy(rows0.at[pl.ds(0, NPS - (NPS // K) * K)],
                    acc.at[pl.ds(sid * NPS + (NPS // K) * K,
                                 NPS - (NPS // K) * K)])

    @pl.when(sid == 0)
    def _():
        pltpu.sync_copy(rows0.at[pl.ds(0, NTAIL)],
                        acc.at[pl.ds(NPS * NS, NTAIL)])

    plsc.subcore_barrier()

    ebuf = (eb0, eb1, eb2)
    semE = (semE0, semE1, semE2)
    rows = (rows0, rows1)
    semG = (semG0, semG1)

    def eload_start(r, e):
        pltpu.async_copy(ei_hbm.at[:, pl.ds((cbase + r) * K, K)],
                         ebuf[e], semE[e])

    def eload_wait(r, e):
        pltpu.make_async_copy(ei_hbm.at[:, pl.ds((cbase + r) * K, K)],
                              ebuf[e], semE[e]).wait()

    def gather_start(e, b):
        pltpu.async_copy(g_hbm.at[ebuf[e].at[0]], rows[b], semG[b])

    def gather_wait(e, b):
        pltpu.make_async_copy(g_hbm.at[ebuf[e].at[0]], rows[b], semG[b]).wait()

    def scatter(e, b):
        pltpu.sync_copy(rows[b], acc.at[ebuf[e].at[1]], add=True)

    eload_start(0, 0)
    eload_start(1, 1)
    eload_start(2, 2)
    eload_wait(0, 0)
    gather_start(0, 0)

    @pl.loop(0, CPW, step=6)
    def _(j):
        for u in range(6):
            r = j + u
            e = u % 3
            b = u % 2
            e1 = (u + 1) % 3

            @pl.when(r + 1 < CPW)
            def _():
                eload_wait(r + 1, e1)
                gather_start(e1, 1 - b)

            gather_wait(e, b)
            scatter(e, b)

            @pl.when(r + 3 < CPW)
            def _():
                eload_start(r + 3, e)

    @pl.when(wid < EXTRA)
    def _():
        pltpu.sync_copy(ei_hbm.at[:, pl.ds((CPW * NW + wid) * K, K)], eb0)
        pltpu.async_copy(g_hbm.at[eb0.at[0]], rows0, semG0)
        pltpu.make_async_copy(g_hbm.at[eb0.at[0]], rows0, semG0).wait()
        pltpu.sync_copy(rows0, acc.at[eb0.at[1]], add=True)

    plsc.subcore_barrier()
    pltpu.sync_copy(acc.at[pl.ds(sid * NPS, NPS)],
                    out_hbm.at[cid, pl.ds(sid * NPS, NPS)])

    @pl.when(sid == 0)
    def _():
        pltpu.sync_copy(acc.at[pl.ds(NPS * NS, NTAIL)],
                        out_hbm.at[cid, pl.ds(NPS * NS, NTAIL)])


_RB = 1000
_GRID = N // _RB


def _tc1a_body(x_ref, wc_ref, h_ref):
    h_ref[...] = jnp.dot(x_ref[...], wc_ref[...],
                         preferred_element_type=jnp.float32)


def _tc1a(x, wc):
    return pl.pallas_call(
        _tc1a_body,
        grid=(_GRID,),
        in_specs=[
            pl.BlockSpec((_RB, C), lambda i: (i, 0)),
            pl.BlockSpec((C, C), lambda i: (0, 0)),
        ],
        out_specs=pl.BlockSpec((_RB, C), lambda i: (i, 0)),
        out_shape=jax.ShapeDtypeStruct((N, C), jnp.float32),
    )(x, wc)


def _tc1b_body(parts_ref, h_ref, g_ref, dinv_ref):
    deg = jnp.sum(parts_ref[...], axis=0) + 1.0
    dinv = lax.rsqrt(deg)
    g_ref[...] = (h_ref[...] * dinv[:, None]).astype(jnp.bfloat16)
    dinv_ref[...] = dinv[:, None]


def _tc1b(parts, h):
    return pl.pallas_call(
        _tc1b_body,
        out_shape=(
            jax.ShapeDtypeStruct((N, C), jnp.bfloat16),
            jax.ShapeDtypeStruct((N, 1), jnp.float32),
        ),
    )(parts, h)


def _tc2_body(s_ref, g_ref, dinv_ref, bc_ref, wl_ref, bl_ref, out_ref):
    dinv = dinv_ref[...]
    ssum = (s_ref[0].astype(jnp.float32) + s_ref[1].astype(jnp.float32)
            + g_ref[...].astype(jnp.float32))
    agg = ssum * dinv + bc_ref[...]
    out = lax.dot_general(agg, wl_ref[...], (((1,), (1,)), ((), ())),
                          preferred_element_type=jnp.float32)
    out_ref[...] = out + bl_ref[...]


def _tc2(s, g, dinv, bc, wl, bl):
    return pl.pallas_call(
        _tc2_body,
        grid=(_GRID,),
        in_specs=[
            pl.BlockSpec((2, _RB, C), lambda i: (0, i, 0)),
            pl.BlockSpec((_RB, C), lambda i: (i, 0)),
            pl.BlockSpec((_RB, 1), lambda i: (i, 0)),
            pl.BlockSpec((1, C), lambda i: (0, 0)),
            pl.BlockSpec((C, C), lambda i: (0, 0)),
            pl.BlockSpec((1, C), lambda i: (0, 0)),
        ],
        out_specs=pl.BlockSpec((_RB, C), lambda i: (i, 0)),
        out_shape=jax.ShapeDtypeStruct((N, C), jnp.float32),
    )(s, g, dinv, bc, wl, bl)


def kernel(x, edge_index, W_conv, b_conv, W_lin, b_lin):
    ei = edge_index.astype(jnp.int32)
    parts = _sc_hist(ei)
    h = _tc1a(x, W_conv)
    g, dinv = _tc1b(parts, h)
    s = _sc_scatter(ei, g)
    return _tc2(s, g, dinv, b_conv.reshape(1, C), W_lin, b_lin.reshape(1, C))


# speedup vs baseline: 57.4377x; 1.0921x over previous
import functools

import jax
import jax.numpy as jnp
from jax import lax
from jax.experimental import pallas as pl
from jax.experimental.pallas import tpu as pltpu
from jax.experimental.pallas import tpu_sc as plsc

N = 10000
E = 320000
C = 128

try:
    _info = plsc.get_sparse_core_info()
    NC, NS = _info.num_cores, _info.num_subcores
except Exception:
    NC, NS = 2, 16
NW = NC * NS
K = 128
CHUNKS = E // K
CPW = CHUNKS // NW
EXTRA = CHUNKS - CPW * NW
NPS = (N // NS) // 8 * 8
NTAIL = N - NPS * NS

_mesh = plsc.VectorSubcoreMesh(core_axis_name="c", subcore_axis_name="s")


@functools.partial(
    pl.kernel,
    out_type=jax.ShapeDtypeStruct((NW, N), jnp.float32),
    mesh=_mesh,
    scratch_types=[
        pltpu.VMEM((2, CPW * K), jnp.int32),
        pltpu.VMEM((2, K), jnp.int32),
        pltpu.VMEM((N,), jnp.float32),
        pltpu.SemaphoreType.DMA,
    ],
    compiler_params=pltpu.CompilerParams(needs_layout_passes=False),
)
def _sc_hist(ei_hbm, out_hbm, ebuf, ebuf_x, hist_v, sem):
    cid = lax.axis_index("c")
    sid = lax.axis_index("s")
    wid = sid * NC + cid
    cp = pltpu.async_copy(ei_hbm.at[:, pl.ds(wid * CPW * K, CPW * K)],
                          ebuf, sem)

    def zbody(i, carry):
        hist_v[pl.ds(i * 16, 16)] = jnp.zeros((16,), jnp.float32)
        return carry

    lax.fori_loop(0, N // 16, zbody, 0)
    cp.wait()
    ones = jnp.ones((16,), jnp.float32)

    def body(i, carry):
        idx = ebuf[1, pl.ds(i * 16, 16)]
        plsc.addupdate_scatter(hist_v, [idx], ones)
        return carry

    lax.fori_loop(0, CPW * K // 16, body, 0)

    @pl.when(wid < EXTRA)
    def _():
        pltpu.sync_copy(ei_hbm.at[:, pl.ds((CPW * NW + wid) * K, K)], ebuf_x)

        def xbody(i, carry):
            idx = ebuf_x[1, pl.ds(i * 16, 16)]
            plsc.addupdate_scatter(hist_v, [idx], ones)
            return carry

        lax.fori_loop(0, K // 16, xbody, 0)

    pltpu.sync_copy(hist_v, out_hbm.at[wid])


@functools.partial(
    pl.kernel,
    out_type=jax.ShapeDtypeStruct((NC, N, C), jnp.bfloat16),
    mesh=_mesh,
    scratch_types=[
        pltpu.VMEM((2, K), jnp.int32),
        pltpu.VMEM((2, K), jnp.int32),
        pltpu.VMEM((2, K), jnp.int32),
        pltpu.VMEM((2, K), jnp.int32),
        pltpu.VMEM((2, K), jnp.int32),
        pltpu.VMEM((2, K), jnp.int32),
        pltpu.VMEM((K, C), jnp.bfloat16),
        pltpu.VMEM((K, C), jnp.bfloat16),
        pltpu.VMEM((K, C), jnp.bfloat16),
        pltpu.VMEM_SHARED((N, C), jnp.bfloat16),
        pltpu.SemaphoreType.DMA,
        pltpu.SemaphoreType.DMA,
        pltpu.SemaphoreType.DMA,
        pltpu.SemaphoreType.DMA,
        pltpu.SemaphoreType.DMA,
        pltpu.SemaphoreType.DMA,
        pltpu.SemaphoreType.DMA,
        pltpu.SemaphoreType.DMA,
        pltpu.SemaphoreType.DMA,
    ],
    compiler_params=pltpu.CompilerParams(use_tc_tiling_on_sc=False),
)
def _sc_scatter(ei_hbm, g_hbm, out_hbm,
                eb0, eb1, eb2, eb3, eb4, eb5, rows0, rows1, rows2, acc,
                semE0, semE1, semE2, semE3, semE4, semE5,
                semG0, semG1, semG2):
    cid = lax.axis_index("c")
    sid = lax.axis_index("s")
    wid = sid * NC + cid
    cbase = wid * CPW

    def zr(r, carry):
        def zc(c, carry2):
            rows0[r, pl.ds(c * 32, 32)] = jnp.zeros((32,), jnp.bfloat16)
            return carry2
        return lax.fori_loop(0, C // 32, zc, carry)

    lax.fori_loop(0, K, zr, 0)
    for q in range(NPS // K):
        pltpu.sync_copy(rows0, acc.at[pl.ds(sid * NPS + q * K, K)])
    pltpu.sync_copy(rows0.at[pl.ds(0, NPS - (NPS // K) * K)],
                    acc.at[pl.ds(sid * NPS + (NPS // K) * K,
                                 NPS - (NPS // K) * K)])

    @pl.when(sid == 0)
    def _():
        pltpu.sync_copy(rows0.at[pl.ds(0, NTAIL)],
                        acc.at[pl.ds(NPS * NS, NTAIL)])

    plsc.subcore_barrier()

    ebuf = (eb0, eb1, eb2, eb3, eb4, eb5)
    semE = (semE0, semE1, semE2, semE3, semE4, semE5)
    rows = (rows0, rows1, rows2)
    semG = (semG0, semG1, semG2)

    def eload_start(r, e):
        pltpu.async_copy(ei_hbm.at[:, pl.ds((cbase + r) * K, K)],
                         ebuf[e], semE[e])

    def eload_wait(r, e):
        pltpu.make_async_copy(ei_hbm.at[:, pl.ds((cbase + r) * K, K)],
                              ebuf[e], semE[e]).wait()

    def gather_start(e, b):
        pltpu.async_copy(g_hbm.at[ebuf[e].at[0]], rows[b], semG[b])

    def gather_wait(e, b):
        pltpu.make_async_copy(g_hbm.at[ebuf[e].at[0]], rows[b], semG[b]).wait()

    def scatter(e, b):
        pltpu.sync_copy(rows[b], acc.at[ebuf[e].at[1]], add=True)

    for p in range(6):
        eload_start(p, p)
    eload_wait(0, 0)
    gather_start(0, 0)
    eload_wait(1, 1)
    gather_start(1, 1)

    @pl.loop(0, CPW, step=6)
    def _(j):
        for u in range(6):
            r = j + u
            e = u % 6
            b = u % 3
            e2 = (u + 2) % 6
            b2 = (u + 2) % 3

            @pl.when(r + 2 < CPW)
            def _():
                eload_wait(r + 2, e2)
                gather_start(e2, b2)

            gather_wait(e, b)
            scatter(e, b)

            @pl.when(r + 6 < CPW)
            def _():
                eload_start(r + 6, e)

    @pl.when(wid < EXTRA)
    def _():
        pltpu.sync_copy(ei_hbm.at[:, pl.ds((CPW * NW + wid) * K, K)], eb0)
        pltpu.async_copy(g_hbm.at[eb0.at[0]], rows0, semG0)
        pltpu.make_async_copy(g_hbm.at[eb0.at[0]], rows0, semG0).wait()
        pltpu.sync_copy(rows0, acc.at[eb0.at[1]], add=True)

    plsc.subcore_barrier()
    pltpu.sync_copy(acc.at[pl.ds(sid * NPS, NPS)],
                    out_hbm.at[cid, pl.ds(sid * NPS, NPS)])

    @pl.when(sid == 0)
    def _():
        pltpu.sync_copy(acc.at[pl.ds(NPS * NS, NTAIL)],
                        out_hbm.at[cid, pl.ds(NPS * NS, NTAIL)])


_RB = 1000
_GRID = N // _RB


def _tc1a_body(x_ref, wc_ref, h_ref):
    h_ref[...] = jnp.dot(x_ref[...], wc_ref[...],
                         preferred_element_type=jnp.float32)


def _tc1a(x, wc):
    return pl.pallas_call(
        _tc1a_body,
        grid=(_GRID,),
        in_specs=[
            pl.BlockSpec((_RB, C), lambda i: (i, 0)),
            pl.BlockSpec((C, C), lambda i: (0, 0)),
        ],
        out_specs=pl.BlockSpec((_RB, C), lambda i: (i, 0)),
        out_shape=jax.ShapeDtypeStruct((N, C), jnp.float32),
    )(x, wc)


def _tc1b_body(parts_ref, h_ref, g_ref, dinv_ref):
    deg = jnp.sum(parts_ref[...], axis=0) + 1.0
    dinv = lax.rsqrt(deg)
    g_ref[...] = (h_ref[...] * dinv[:, None]).astype(jnp.bfloat16)
    dinv_ref[...] = dinv[:, None]


def _tc1b(parts, h):
    return pl.pallas_call(
        _tc1b_body,
        out_shape=(
            jax.ShapeDtypeStruct((N, C), jnp.bfloat16),
            jax.ShapeDtypeStruct((N, 1), jnp.float32),
        ),
    )(parts, h)


def _tc2_body(s_ref, g_ref, dinv_ref, bc_ref, wl_ref, bl_ref, out_ref):
    dinv = dinv_ref[...]
    ssum = (s_ref[0].astype(jnp.float32) + s_ref[1].astype(jnp.float32)
            + g_ref[...].astype(jnp.float32))
    agg = ssum * dinv + bc_ref[...]
    out = lax.dot_general(agg, wl_ref[...], (((1,), (1,)), ((), ())),
                          preferred_element_type=jnp.float32)
    out_ref[...] = out + bl_ref[...]


def _tc2(s, g, dinv, bc, wl, bl):
    return pl.pallas_call(
        _tc2_body,
        grid=(_GRID,),
        in_specs=[
            pl.BlockSpec((2, _RB, C), lambda i: (0, i, 0)),
            pl.BlockSpec((_RB, C), lambda i: (i, 0)),
            pl.BlockSpec((_RB, 1), lambda i: (i, 0)),
            pl.BlockSpec((1, C), lambda i: (0, 0)),
            pl.BlockSpec((C, C), lambda i: (0, 0)),
            pl.BlockSpec((1, C), lambda i: (0, 0)),
        ],
        out_specs=pl.BlockSpec((_RB, C), lambda i: (i, 0)),
        out_shape=jax.ShapeDtypeStruct((N, C), jnp.float32),
    )(s, g, dinv, bc, wl, bl)


def kernel(x, edge_index, W_conv, b_conv, W_lin, b_lin):
    ei = edge_index.astype(jnp.int32)
    parts = _sc_hist(ei)
    h = _tc1a(x, W_conv)
    g, dinv = _tc1b(parts, h)
    s = _sc_scatter(ei, g)
    return _tc2(s, g, dinv, b_conv.reshape(1, C), W_lin, b_lin.reshape(1, C))
